# baseline (speedup 1.0000x reference)
import functools

import jax
import jax.numpy as jnp
from jax.experimental import pallas as pl
from jax.experimental.pallas import tpu as pltpu

N = 20000
C = 81
P = 128
BLK = 2000
GRID = N // BLK


def _body(prob_ref, lab_ref, w_ref, pcl_ref, pcp_ref, imw_ref, iml_ref,
          out_ref):
    i = pl.program_id(0)

    block = prob_ref[...]
    e0 = (jax.lax.broadcasted_iota(jnp.int32, (1, C), 1) == 0).astype(
        jnp.float32)
    col = jax.lax.dot_general(
        e0, block, (((1,), (1,)), ((), ())),
        preferred_element_type=jnp.float32)
    lab = lab_ref[0]
    w = w_ref[0]
    bg_active = (iml_ref[0, 0] != 0.0).astype(jnp.float32)
    mask = (lab == 0).astype(jnp.float32)
    bg_part = -bg_active * jnp.sum(mask * w * jnp.log(col), axis=(0, 1),
                                   keepdims=True)

    @pl.when(i == 0)
    def _init():
        pcl = pcl_ref[...]
        iota_c = jax.lax.broadcasted_iota(jnp.int32, (C, P), 0)
        onehot = (iota_c == pcl).astype(jnp.float32)
        gathered = jax.lax.dot_general(
            iml_ref[...], onehot, (((1,), (0,)), ((), ())),
            preferred_element_type=jnp.float32)
        fg_active = (gathered != 0.0) & (pcl > 0)
        fg_vals = imw_ref[...] * jnp.log(pcp_ref[...])
        fg = -jnp.sum(jnp.where(fg_active, fg_vals, 0.0), axis=(0, 1),
                      keepdims=True)
        out_ref[...] = fg + bg_part

    @pl.when(i > 0)
    def _acc():
        out_ref[...] += bg_part

    @pl.when(i == GRID - 1)
    def _fin():
        out_ref[...] = out_ref[...] * (1.0 / N)


@functools.partial(jax.jit, static_argnames=())
def kernel(pcl_prob, labels, cls_loss_weights, gt_assignment, pc_labels,
           pc_probs, pc_count, img_cls_loss_weights, im_labels_real):
    del gt_assignment, pc_count
    lab3 = labels.reshape(GRID, 1, BLK)
    w3 = cls_loss_weights.reshape(GRID, 1, BLK)
    out = pl.pallas_call(
        _body,
        grid=(GRID,),
        in_specs=[
            pl.BlockSpec((BLK, C), lambda i: (i, 0)),
            pl.BlockSpec((1, 1, BLK), lambda i: (i, 0, 0)),
            pl.BlockSpec((1, 1, BLK), lambda i: (i, 0, 0)),
            pl.BlockSpec((1, P), lambda i: (0, 0)),
            pl.BlockSpec((1, P), lambda i: (0, 0)),
            pl.BlockSpec((1, P), lambda i: (0, 0)),
            pl.BlockSpec((1, C), lambda i: (0, 0)),
        ],
        out_specs=pl.BlockSpec((1, 1), lambda i: (0, 0)),
        out_shape=jax.ShapeDtypeStruct((1, 1), jnp.float32),
        compiler_params=pltpu.CompilerParams(
            dimension_semantics=("arbitrary",)),
    )(pcl_prob, lab3, w3, pc_labels.reshape(1, P), pc_probs.reshape(1, P),
      img_cls_loss_weights.reshape(1, P), im_labels_real.reshape(1, C))
    return out[0, 0]


# speedup vs baseline: 1.2136x; 1.2136x over previous
import functools

import jax
import jax.numpy as jnp
from jax.experimental import pallas as pl
from jax.experimental.pallas import tpu as pltpu

N = 20000
C = 81
P = 128


def _body(prob_ref, lab_ref, w_ref, pcl_ref, pcp_ref, imw_ref, iml_ref,
          out_ref):
    block = prob_ref[...]
    e0 = (jax.lax.broadcasted_iota(jnp.int32, (1, C), 1) == 0).astype(
        jnp.float32)
    col = jax.lax.dot_general(
        e0, block, (((1,), (1,)), ((), ())),
        preferred_element_type=jnp.float32)

    lab = lab_ref[...]
    w = w_ref[...]
    bg_active = (iml_ref[0, 0] != 0.0).astype(jnp.float32)
    mask = (lab == 0).astype(jnp.float32)
    bg = -bg_active * jnp.sum(mask * w * jnp.log(col), axis=(0, 1),
                              keepdims=True)

    pcl = pcl_ref[...]
    iota_c = jax.lax.broadcasted_iota(jnp.int32, (C, P), 0)
    onehot = (iota_c == pcl).astype(jnp.float32)
    gathered = jax.lax.dot_general(
        iml_ref[...], onehot, (((1,), (0,)), ((), ())),
        preferred_element_type=jnp.float32)
    fg_active = (gathered != 0.0) & (pcl > 0)
    fg_vals = imw_ref[...] * jnp.log(pcp_ref[...])
    fg = -jnp.sum(jnp.where(fg_active, fg_vals, 0.0), axis=(0, 1),
                  keepdims=True)

    out_ref[...] = (bg + fg) * (1.0 / N)


@functools.partial(jax.jit, static_argnames=())
def kernel(pcl_prob, labels, cls_loss_weights, gt_assignment, pc_labels,
           pc_probs, pc_count, img_cls_loss_weights, im_labels_real):
    del gt_assignment, pc_count
    out = pl.pallas_call(
        _body,
        out_shape=jax.ShapeDtypeStruct((1, 1), jnp.float32),
    )(pcl_prob, labels.reshape(1, N), cls_loss_weights.reshape(1, N),
      pc_labels.reshape(1, P), pc_probs.reshape(1, P),
      img_cls_loss_weights.reshape(1, P), im_labels_real.reshape(1, C))
    return out[0, 0]
